# baseline (speedup 1.0000x reference)
import functools

import jax
import jax.numpy as jnp
from jax import lax
from jax.experimental import pallas as pl
from jax.experimental.pallas import tpu as pltpu
from jax.experimental.pallas import tpu_sc as plsc

NUM_EMB = 1000000
DIM = 32
SEQ = 50
BATCH = 16384
SCALE = 50.0

NC = 2
NS = 16
NW = NC * NS

NB_PER_W = BATCH // NW
BCHUNK = 16
ROWS = BCHUNK * SEQ
NCHUNKS = NB_PER_W // BCHUNK
NPAIRS = NCHUNKS // 2

_mesh = plsc.VectorSubcoreMesh(core_axis_name="c", subcore_axis_name="s")


@functools.partial(
    pl.kernel,
    out_type=jax.ShapeDtypeStruct((SEQ, DIM // 8, BATCH // 128, 8, 128),
                                  jnp.float32),
    mesh=_mesh,
    compiler_params=pltpu.CompilerParams(
        use_tc_tiling_on_sc=False, needs_layout_passes=False
    ),
    scratch_types=[
        pltpu.VMEM((2, ROWS), jnp.int32),
        pltpu.VMEM((2, ROWS, DIM), jnp.float32),
        pltpu.VMEM((2, SEQ, DIM // 8, 8, BCHUNK), jnp.float32),
        pltpu.SemaphoreType.DMA,
        pltpu.SemaphoreType.DMA,
        pltpu.SemaphoreType.DMA,
        pltpu.SemaphoreType.DMA,
    ],
)
def _emb_lookup(idx_hbm, table_hbm, out_hbm, idx_v, rows_v, tout_v,
                g0, g1, o0, o1):
    wid = lax.axis_index("s") * NC + lax.axis_index("c")
    b_base = wid * NB_PER_W
    gsems = (g0, g1)
    osems = (o0, o1)
    lanes50 = lax.iota(jnp.int32, 16) * SEQ

    def fire(buf, c):
        k0 = pl.multiple_of((b_base + c * BCHUNK) * SEQ, ROWS)
        pltpu.sync_copy(idx_hbm.at[pl.ds(k0, ROWS)], idx_v.at[buf])
        pltpu.async_copy(table_hbm.at[idx_v.at[buf]], rows_v.at[buf],
                         gsems[buf])

    def out_slices(c):
        b0 = b_base + c * BCHUNK
        bblk = b0 // 128
        bb0 = pl.multiple_of(b0 % 128, BCHUNK)
        return bblk, bb0

    def process(buf, c):
        pltpu.make_async_copy(table_hbm.at[idx_v.at[buf]], rows_v.at[buf],
                              gsems[buf]).wait()

        @pl.when(c >= 2)
        def _():
            bblk_p, bb0_p = out_slices(c - 2)
            pltpu.make_async_copy(
                tout_v.at[buf],
                out_hbm.at[:, :, bblk_p, :, pl.ds(bb0_p, BCHUNK)],
                osems[buf],
            ).wait()

        rv = rows_v.at[buf]

        @plsc.parallel_loop(0, SEQ * DIM, unroll=16)
        def _(t):
            s = t >> 5
            col = t & 31
            v = plsc.load_gather(
                rv, [lanes50 + s, jnp.full((16,), col, jnp.int32)]
            )
            tout_v[buf, s, (t >> 3) & 3, t & 7] = v * SCALE

        bblk, bb0 = out_slices(c)
        pltpu.async_copy(
            tout_v.at[buf],
            out_hbm.at[:, :, bblk, :, pl.ds(bb0, BCHUNK)],
            osems[buf],
        )

    fire(0, 0)

    def pair_body(i, carry):
        c0 = i * 2
        fire(1, c0 + 1)
        process(0, c0)

        @pl.when(c0 + 2 < NCHUNKS)
        def _():
            fire(0, c0 + 2)

        process(1, c0 + 1)
        return carry

    lax.fori_loop(0, NPAIRS, pair_body, 0)

    for c in (NCHUNKS - 2, NCHUNKS - 1):
        buf = c % 2
        bblk, bb0 = out_slices(c)
        pltpu.make_async_copy(
            tout_v.at[buf],
            out_hbm.at[:, :, bblk, :, pl.ds(bb0, BCHUNK)],
            osems[buf],
        ).wait()


def kernel(input, embedding_weight):
    idx = input.astype(jnp.int32).reshape(BATCH * SEQ)
    o5 = _emb_lookup(idx, embedding_weight)
    return o5.transpose(2, 4, 0, 1, 3).reshape(BATCH, SEQ, DIM)


# speedup vs baseline: 1.0220x; 1.0220x over previous
import functools

import jax
import jax.numpy as jnp
from jax import lax
from jax.experimental import pallas as pl
from jax.experimental.pallas import tpu as pltpu
from jax.experimental.pallas import tpu_sc as plsc

NUM_EMB = 1000000
DIM = 32
SEQ = 50
BATCH = 16384
SCALE = 50.0

NC = 2
NS = 16
NW = NC * NS

NB_PER_W = BATCH // NW
BCHUNK = 16
ROWS = BCHUNK * SEQ
NCHUNKS = NB_PER_W // BCHUNK
NPAIRS = NCHUNKS // 2

_mesh = plsc.VectorSubcoreMesh(core_axis_name="c", subcore_axis_name="s")


@functools.partial(
    pl.kernel,
    out_type=jax.ShapeDtypeStruct((SEQ, DIM // 8, BATCH // 128, 8, 128),
                                  jnp.float32),
    mesh=_mesh,
    compiler_params=pltpu.CompilerParams(
        use_tc_tiling_on_sc=False, needs_layout_passes=False
    ),
    scratch_types=[
        pltpu.VMEM((2, ROWS), jnp.int32),
        pltpu.VMEM((2, ROWS, DIM), jnp.float32),
        pltpu.VMEM((2, SEQ, DIM // 8, 8, BCHUNK), jnp.float32),
        pltpu.SemaphoreType.DMA,
        pltpu.SemaphoreType.DMA,
        pltpu.SemaphoreType.DMA,
        pltpu.SemaphoreType.DMA,
    ],
)
def _emb_lookup(idx_hbm, table_hbm, out_hbm, idx_v, rows_v, tout_v,
                g0, g1, o0, o1):
    wid = lax.axis_index("s") * NC + lax.axis_index("c")
    b_base = wid * NB_PER_W
    gsems = (g0, g1)
    osems = (o0, o1)
    lanes50 = lax.iota(jnp.int32, 16) * SEQ

    def fire(buf, c):
        k0 = pl.multiple_of((b_base + c * BCHUNK) * SEQ, ROWS)
        pltpu.sync_copy(idx_hbm.at[pl.ds(k0, ROWS)], idx_v.at[buf])
        pltpu.async_copy(table_hbm.at[idx_v.at[buf]], rows_v.at[buf],
                         gsems[buf])

    def out_slices(c):
        b0 = b_base + c * BCHUNK
        bblk = b0 // 128
        bb0 = pl.multiple_of(b0 % 128, BCHUNK)
        return bblk, bb0

    def process(buf, c):
        pltpu.make_async_copy(table_hbm.at[idx_v.at[buf]], rows_v.at[buf],
                              gsems[buf]).wait()

        @pl.when(c >= 2)
        def _():
            bblk_p, bb0_p = out_slices(c - 2)
            pltpu.make_async_copy(
                tout_v.at[buf],
                out_hbm.at[:, :, bblk_p, :, pl.ds(bb0_p, BCHUNK)],
                osems[buf],
            ).wait()

        rv = rows_v.at[buf]

        @plsc.parallel_loop(0, SEQ * DIM, unroll=8)
        def _(t):
            s = t >> 5
            col = t & 31
            v = plsc.load_gather(
                rv, [lanes50 + s, jnp.full((16,), col, jnp.int32)]
            )
            tout_v[buf, s, (t >> 3) & 3, t & 7] = v * SCALE

        bblk, bb0 = out_slices(c)
        pltpu.async_copy(
            tout_v.at[buf],
            out_hbm.at[:, :, bblk, :, pl.ds(bb0, BCHUNK)],
            osems[buf],
        )

    fire(0, 0)

    def pair_body(i, carry):
        c0 = i * 2
        fire(1, c0 + 1)
        process(0, c0)

        @pl.when(c0 + 2 < NCHUNKS)
        def _():
            fire(0, c0 + 2)

        process(1, c0 + 1)
        return carry

    lax.fori_loop(0, NPAIRS, pair_body, 0)

    for c in (NCHUNKS - 2, NCHUNKS - 1):
        buf = c % 2
        bblk, bb0 = out_slices(c)
        pltpu.make_async_copy(
            tout_v.at[buf],
            out_hbm.at[:, :, bblk, :, pl.ds(bb0, BCHUNK)],
            osems[buf],
        ).wait()


def kernel(input, embedding_weight):
    idx = input.astype(jnp.int32).reshape(BATCH * SEQ)
    o5 = _emb_lookup(idx, embedding_weight)
    return o5.transpose(2, 4, 0, 1, 3).reshape(BATCH, SEQ, DIM)
